# baseline (device time: 26001 ns/iter reference)
import jax
import jax.numpy as jnp
from jax import lax
from jax.experimental import pallas as pl
from jax.experimental.pallas import tpu as pltpu

N_DEV = 4
N_HOP = N_DEV - 1
S = 4
NSTR = 2 * S


def kernel(x, w_mat):
    m_global, k_per = x.shape
    _, n = w_mat.shape
    m_per = m_global // N_DEV
    nh = n // 2
    ns = nh // S

    def out_lo(k):
        return k * ns if k < S else nh + (k - S) * ns

    def kdir(k):
        return 0 if k < S else 1

    def body(x_hbm, w_hbm, out_hbm, *scratch):
        comm = scratch[0:NSTR]
        ssems = scratch[NSTR:2 * NSTR]
        rsems = scratch[2 * NSTR:3 * NSTR]
        x_ref, w_ref = scratch[3 * NSTR], scratch[3 * NSTR + 1]
        load_sems = scratch[3 * NSTR + 2]
        out_v = scratch[3 * NSTR + 3]
        store_sems = scratch[3 * NSTR + 4]

        p = lax.axis_index("i")
        left = lax.rem(p + N_DEV - 1, N_DEV)
        right = lax.rem(p + 1, N_DEV)

        barrier_sem = pltpu.get_barrier_semaphore()
        for nbr in [left, right]:
            pl.semaphore_signal(
                barrier_sem, inc=1,
                device_id=(nbr,), device_id_type=pl.DeviceIdType.MESH,
            )

        dma_w = pltpu.make_async_copy(w_hbm, w_ref, load_sems.at[0])
        dma_w.start()
        chunk_order = [
            lax.rem(p + N_DEV - 1, N_DEV),
            lax.rem(p + 1, N_DEV),
            lax.rem(p + 2, N_DEV),
            p,
        ]
        dma_x = []
        for i, c in enumerate(chunk_order):
            rows = pl.ds(c * m_per, m_per)
            d = pltpu.make_async_copy(
                x_hbm.at[rows, :], x_ref.at[rows, :], load_sems.at[1 + i]
            )
            d.start()
            dma_x.append(d)

        rdmas = {}
        for t in range(N_HOP):
            for k in range(NSTR):
                rdmas[(t, k)] = pltpu.make_async_remote_copy(
                    src_ref=comm[k].at[t],
                    dst_ref=comm[k].at[t + 1],
                    send_sem=ssems[k].at[t],
                    recv_sem=rsems[k].at[t],
                    device_id=(right if kdir(k) == 0 else left,),
                    device_id_type=pl.DeviceIdType.MESH,
                )

        def partial_r(c):
            return jnp.dot(
                x_ref[pl.ds(c * m_per, m_per), :], w_ref[:, pl.ds(0, nh)],
                preferred_element_type=jnp.float32,
            )

        def partial_l(c):
            return jnp.dot(
                x_ref[pl.ds(c * m_per, m_per), :], w_ref[:, pl.ds(nh, nh)],
                preferred_element_type=jnp.float32,
            )

        dma_w.wait()
        dma_x[0].wait()
        dma_x[1].wait()
        barrier_waited = False
        for s in range(S):
            for k, c in ((s, chunk_order[0]), (S + s, chunk_order[1])):
                comm[k][0, :, :] = jnp.dot(
                    x_ref[pl.ds(c * m_per, m_per), :],
                    w_ref[:, pl.ds(out_lo(k), ns)],
                    preferred_element_type=jnp.float32,
                )
                if not barrier_waited:
                    pl.semaphore_wait(barrier_sem, 2)
                    barrier_waited = True
                rdmas[(0, k)].start()

        out_dmas = []
        for t in range(N_HOP):
            if t == 0:
                dma_x[2].wait()
            elif t == N_HOP - 1:
                dma_x[3].wait()
            part_r = partial_r(lax.rem(p + 2 * N_DEV - 2 - t, N_DEV))
            part_l = partial_l(lax.rem(p + 2 + t, N_DEV))
            for s in range(S):
                for k in (s, S + s):
                    part = (part_r if k < S else part_l)[
                        :, (k % S) * ns:(k % S + 1) * ns
                    ]
                    rdmas[(t, k)].wait_recv()
                    acc = comm[k][t + 1, :, :] + part
                    if t < N_HOP - 1:
                        comm[k][t + 1, :, :] = acc
                        rdmas[(t + 1, k)].start()
                    else:
                        out_v[:, pl.ds(out_lo(k), ns)] = acc
                        d = pltpu.make_async_copy(
                            out_v.at[:, pl.ds(out_lo(k), ns)],
                            out_hbm.at[:, pl.ds(out_lo(k), ns)],
                            store_sems.at[k],
                        )
                        d.start()
                        out_dmas.append(d)

        for t in range(N_HOP):
            for k in range(NSTR):
                rdmas[(t, k)].wait_send()
        for d in out_dmas:
            d.wait()

    return pl.pallas_call(
        body,
        out_shape=jax.ShapeDtypeStruct((m_per, n), jnp.float32),
        in_specs=[
            pl.BlockSpec(memory_space=pltpu.MemorySpace.HBM),
            pl.BlockSpec(memory_space=pltpu.MemorySpace.HBM),
        ],
        out_specs=pl.BlockSpec(memory_space=pltpu.MemorySpace.HBM),
        scratch_shapes=(
            [pltpu.VMEM((N_HOP + 1, m_per, ns), jnp.float32)] * NSTR
            + [pltpu.SemaphoreType.DMA((N_HOP,))] * (2 * NSTR)
            + [
                pltpu.VMEM((m_global, k_per), jnp.float32),
                pltpu.VMEM((k_per, n), jnp.float32),
                pltpu.SemaphoreType.DMA((5,)),
                pltpu.VMEM((m_per, n), jnp.float32),
                pltpu.SemaphoreType.DMA((NSTR,)),
            ]
        ),
        compiler_params=pltpu.CompilerParams(collective_id=0),
    )(x, w_mat)


# device time: 25754 ns/iter; 1.0096x vs baseline; 1.0096x over previous
import jax
import jax.numpy as jnp
from jax import lax
from jax.experimental import pallas as pl
from jax.experimental.pallas import tpu as pltpu

N_DEV = 4
N_HOP = N_DEV - 1
S = 2
NSTR = 2 * S


def kernel(x, w_mat):
    m_global, k_per = x.shape
    _, n = w_mat.shape
    m_per = m_global // N_DEV
    nh = n // 2
    ns = nh // S

    def out_lo(k):
        return k * ns if k < S else nh + (k - S) * ns

    def kdir(k):
        return 0 if k < S else 1

    def body(x_hbm, w_hbm, out_hbm, *scratch):
        comm = scratch[0:NSTR]
        ssems = scratch[NSTR:2 * NSTR]
        rsems = scratch[2 * NSTR:3 * NSTR]
        x_ref, w_ref = scratch[3 * NSTR], scratch[3 * NSTR + 1]
        load_sems = scratch[3 * NSTR + 2]
        out_v = scratch[3 * NSTR + 3]
        store_sems = scratch[3 * NSTR + 4]

        p = lax.axis_index("i")
        left = lax.rem(p + N_DEV - 1, N_DEV)
        right = lax.rem(p + 1, N_DEV)

        barrier_sem = pltpu.get_barrier_semaphore()
        for nbr in [left, right]:
            pl.semaphore_signal(
                barrier_sem, inc=1,
                device_id=(nbr,), device_id_type=pl.DeviceIdType.MESH,
            )

        dma_w = pltpu.make_async_copy(w_hbm, w_ref, load_sems.at[0])
        dma_w.start()
        chunk_order = [
            lax.rem(p + N_DEV - 1, N_DEV),
            lax.rem(p + 1, N_DEV),
            lax.rem(p + 2, N_DEV),
            p,
        ]
        dma_x = []
        for i, c in enumerate(chunk_order):
            rows = pl.ds(c * m_per, m_per)
            d = pltpu.make_async_copy(
                x_hbm.at[rows, :], x_ref.at[rows, :], load_sems.at[1 + i]
            )
            d.start()
            dma_x.append(d)

        rdmas = {}
        for t in range(N_HOP):
            for k in range(NSTR):
                rdmas[(t, k)] = pltpu.make_async_remote_copy(
                    src_ref=comm[k].at[t],
                    dst_ref=comm[k].at[t + 1],
                    send_sem=ssems[k].at[t],
                    recv_sem=rsems[k].at[t],
                    device_id=(right if kdir(k) == 0 else left,),
                    device_id_type=pl.DeviceIdType.MESH,
                )

        def partial_r(c):
            return jnp.dot(
                x_ref[pl.ds(c * m_per, m_per), :], w_ref[:, pl.ds(0, nh)],
                preferred_element_type=jnp.float32,
            )

        def partial_l(c):
            return jnp.dot(
                x_ref[pl.ds(c * m_per, m_per), :], w_ref[:, pl.ds(nh, nh)],
                preferred_element_type=jnp.float32,
            )

        dma_w.wait()
        dma_x[0].wait()
        dma_x[1].wait()
        barrier_waited = False
        for s in range(S):
            for k, c in ((s, chunk_order[0]), (S + s, chunk_order[1])):
                comm[k][0, :, :] = jnp.dot(
                    x_ref[pl.ds(c * m_per, m_per), :],
                    w_ref[:, pl.ds(out_lo(k), ns)],
                    preferred_element_type=jnp.float32,
                )
                if not barrier_waited:
                    pl.semaphore_wait(barrier_sem, 2)
                    barrier_waited = True
                rdmas[(0, k)].start()

        out_dmas = []
        for t in range(N_HOP):
            if t == 0:
                dma_x[2].wait()
            elif t == N_HOP - 1:
                dma_x[3].wait()
            part_r = partial_r(lax.rem(p + 2 * N_DEV - 2 - t, N_DEV))
            part_l = partial_l(lax.rem(p + 2 + t, N_DEV))
            for s in range(S):
                for k in (s, S + s):
                    part = (part_r if k < S else part_l)[
                        :, (k % S) * ns:(k % S + 1) * ns
                    ]
                    rdmas[(t, k)].wait_recv()
                    acc = comm[k][t + 1, :, :] + part
                    if t < N_HOP - 1:
                        comm[k][t + 1, :, :] = acc
                        rdmas[(t + 1, k)].start()
                    else:
                        out_v[:, pl.ds(out_lo(k), ns)] = acc
                        d = pltpu.make_async_copy(
                            out_v.at[:, pl.ds(out_lo(k), ns)],
                            out_hbm.at[:, pl.ds(out_lo(k), ns)],
                            store_sems.at[k],
                        )
                        d.start()
                        out_dmas.append(d)

        for t in range(N_HOP):
            for k in range(NSTR):
                rdmas[(t, k)].wait_send()
        for d in out_dmas:
            d.wait()

    return pl.pallas_call(
        body,
        out_shape=jax.ShapeDtypeStruct((m_per, n), jnp.float32),
        in_specs=[
            pl.BlockSpec(memory_space=pltpu.MemorySpace.HBM),
            pl.BlockSpec(memory_space=pltpu.MemorySpace.HBM),
        ],
        out_specs=pl.BlockSpec(memory_space=pltpu.MemorySpace.HBM),
        scratch_shapes=(
            [pltpu.VMEM((N_HOP + 1, m_per, ns), jnp.float32)] * NSTR
            + [pltpu.SemaphoreType.DMA((N_HOP,))] * (2 * NSTR)
            + [
                pltpu.VMEM((m_global, k_per), jnp.float32),
                pltpu.VMEM((k_per, n), jnp.float32),
                pltpu.SemaphoreType.DMA((5,)),
                pltpu.VMEM((m_per, n), jnp.float32),
                pltpu.SemaphoreType.DMA((NSTR,)),
            ]
        ),
        compiler_params=pltpu.CompilerParams(collective_id=0),
    )(x, w_mat)


# device time: 25707 ns/iter; 1.0114x vs baseline; 1.0018x over previous
import jax
import jax.numpy as jnp
from jax import lax
from jax.experimental import pallas as pl
from jax.experimental.pallas import tpu as pltpu

N_DEV = 4
N_HOP = N_DEV - 1
S = 2
NSTR = 2 * S


def kernel(x, w_mat):
    m_global, k_per = x.shape
    _, n = w_mat.shape
    m_per = m_global // N_DEV
    nh = n // 2
    ns = nh // S

    def out_lo(k):
        return k * ns if k < S else nh + (k - S) * ns

    def kdir(k):
        return 0 if k < S else 1

    def body(x_ref, w_ref, out_hbm, *scratch):
        comm = scratch[0:NSTR]
        ssems = scratch[NSTR:2 * NSTR]
        rsems = scratch[2 * NSTR:3 * NSTR]
        out_v = scratch[3 * NSTR]
        store_sems = scratch[3 * NSTR + 1]

        p = lax.axis_index("i")
        left = lax.rem(p + N_DEV - 1, N_DEV)
        right = lax.rem(p + 1, N_DEV)

        barrier_sem = pltpu.get_barrier_semaphore()
        for nbr in [left, right]:
            pl.semaphore_signal(
                barrier_sem, inc=1,
                device_id=(nbr,), device_id_type=pl.DeviceIdType.MESH,
            )

        chunk_order = [
            lax.rem(p + N_DEV - 1, N_DEV),
            lax.rem(p + 1, N_DEV),
            lax.rem(p + 2, N_DEV),
            p,
        ]

        rdmas = {}
        for t in range(N_HOP):
            for k in range(NSTR):
                rdmas[(t, k)] = pltpu.make_async_remote_copy(
                    src_ref=comm[k].at[t],
                    dst_ref=comm[k].at[t + 1],
                    send_sem=ssems[k].at[t],
                    recv_sem=rsems[k].at[t],
                    device_id=(right if kdir(k) == 0 else left,),
                    device_id_type=pl.DeviceIdType.MESH,
                )

        def partial_r(c):
            return jnp.dot(
                x_ref[pl.ds(c * m_per, m_per), :], w_ref[:, pl.ds(0, nh)],
                preferred_element_type=jnp.float32,
            )

        def partial_l(c):
            return jnp.dot(
                x_ref[pl.ds(c * m_per, m_per), :], w_ref[:, pl.ds(nh, nh)],
                preferred_element_type=jnp.float32,
            )

        barrier_waited = False
        for s in range(S):
            for k, c in ((s, chunk_order[0]), (S + s, chunk_order[1])):
                comm[k][0, :, :] = jnp.dot(
                    x_ref[pl.ds(c * m_per, m_per), :],
                    w_ref[:, pl.ds(out_lo(k), ns)],
                    preferred_element_type=jnp.float32,
                )
                if not barrier_waited:
                    pl.semaphore_wait(barrier_sem, 2)
                    barrier_waited = True
                rdmas[(0, k)].start()

        out_dmas = []
        for t in range(N_HOP):
            part_r = partial_r(lax.rem(p + 2 * N_DEV - 2 - t, N_DEV))
            part_l = partial_l(lax.rem(p + 2 + t, N_DEV))
            for s in range(S):
                for k in (s, S + s):
                    part = (part_r if k < S else part_l)[
                        :, (k % S) * ns:(k % S + 1) * ns
                    ]
                    rdmas[(t, k)].wait_recv()
                    acc = comm[k][t + 1, :, :] + part
                    if t < N_HOP - 1:
                        comm[k][t + 1, :, :] = acc
                        rdmas[(t + 1, k)].start()
                    else:
                        out_v[:, pl.ds(out_lo(k), ns)] = acc
                        d = pltpu.make_async_copy(
                            out_v.at[:, pl.ds(out_lo(k), ns)],
                            out_hbm.at[:, pl.ds(out_lo(k), ns)],
                            store_sems.at[k],
                        )
                        d.start()
                        out_dmas.append(d)

        for t in range(N_HOP):
            for k in range(NSTR):
                rdmas[(t, k)].wait_send()
        for d in out_dmas:
            d.wait()

    return pl.pallas_call(
        body,
        out_shape=jax.ShapeDtypeStruct((m_per, n), jnp.float32),
        in_specs=[
            pl.BlockSpec(memory_space=pltpu.VMEM),
            pl.BlockSpec(memory_space=pltpu.VMEM),
        ],
        out_specs=pl.BlockSpec(memory_space=pltpu.MemorySpace.HBM),
        scratch_shapes=(
            [pltpu.VMEM((N_HOP + 1, m_per, ns), jnp.float32)] * NSTR
            + [pltpu.SemaphoreType.DMA((N_HOP,))] * (2 * NSTR)
            + [
                pltpu.VMEM((m_per, n), jnp.float32),
                pltpu.SemaphoreType.DMA((NSTR,)),
            ]
        ),
        compiler_params=pltpu.CompilerParams(collective_id=0),
    )(x, w_mat)
